# baseline (speedup 1.0000x reference)
import functools
import math

import jax
import jax.numpy as jnp
from jax import lax
from jax.experimental import pallas as pl
from jax.experimental.pallas import tpu as pltpu
from jax.experimental.pallas import tpu_sc as plsc

D_MODEL = 64
SCALE = math.sqrt(D_MODEL)

NC = 2
NS = 16
NW = NC * NS
L = 16

CHUNK = 128
NBUF = 4


def _make_gather(B: int):
    assert B % (NW * CHUNK) == 0
    b_per_w = B // NW
    n_chunks = b_per_w // CHUNK
    assert n_chunks % NBUF == 0 and n_chunks // NBUF >= 2
    vecs_per_row = D_MODEL // L

    mesh = plsc.VectorSubcoreMesh(core_axis_name="c", subcore_axis_name="s")

    @functools.partial(
        pl.kernel,
        out_type=jax.ShapeDtypeStruct((B, D_MODEL), jnp.float32),
        mesh=mesh,
        compiler_params=pltpu.CompilerParams(use_tc_tiling_on_sc=False),
        scratch_types=[
            pltpu.VMEM((b_per_w,), jnp.int32),
            [pltpu.VMEM((CHUNK, D_MODEL), jnp.float32)] * NBUF,
            [pltpu.VMEM((CHUNK, D_MODEL), jnp.float32)] * NBUF,
            [pltpu.SemaphoreType.DMA] * NBUF,
            [pltpu.SemaphoreType.DMA] * NBUF,
        ],
    )
    def gather_scale(x_hbm, table_hbm, out_hbm, idx_v, gbufs, wbufs, gsems, wsems):
        wid = lax.axis_index("s") * NC + lax.axis_index("c")
        base = wid * b_per_w
        pltpu.sync_copy(x_hbm.at[pl.ds(base, b_per_w)], idx_v)

        def start_gather(b, j):
            pltpu.async_copy(
                table_hbm.at[idx_v.at[pl.ds(j * CHUNK, CHUNK)]], gbufs[b], gsems[b]
            )

        def gather_done(b):
            pltpu.make_async_copy(
                table_hbm.at[pl.ds(0, CHUNK)], gbufs[b], gsems[b]
            ).wait()

        def write_done(b):
            pltpu.make_async_copy(
                wbufs[b], out_hbm.at[pl.ds(0, CHUNK)], wsems[b]
            ).wait()

        for b in range(NBUF):
            start_gather(b, b)

        @pl.loop(0, n_chunks - NBUF, step=NBUF)
        def _outer(j0):
            for b in range(NBUF):
                j = j0 + b
                gather_done(b)

                @pl.when(j0 > 0)
                def _():
                    write_done(b)

                @pl.loop(0, CHUNK, unroll=8)
                def _row(r):
                    for v in range(vecs_per_row):
                        sl = pl.ds(v * L, L)
                        wbufs[b][r, sl] = gbufs[b][r, sl] * SCALE

                pltpu.async_copy(
                    wbufs[b], out_hbm.at[pl.ds(base + j * CHUNK, CHUNK)], wsems[b]
                )
                start_gather(b, j + NBUF)

        for b in range(NBUF):
            j = n_chunks - NBUF + b
            gather_done(b)
            write_done(b)

            @pl.loop(0, CHUNK, unroll=8)
            def _row(r):
                for v in range(vecs_per_row):
                    sl = pl.ds(v * L, L)
                    wbufs[b][r, sl] = gbufs[b][r, sl] * SCALE

            pltpu.async_copy(
                wbufs[b], out_hbm.at[pl.ds(base + j * CHUNK, CHUNK)], wsems[b]
            )
        for b in range(NBUF):
            write_done(b)

    return gather_scale


@jax.jit
def kernel(x, table):
    shape = x.shape
    flat = x.reshape(-1)
    out = _make_gather(flat.shape[0])(flat, table)
    return out.reshape(shape + (D_MODEL,))


# speedup vs baseline: 1.2680x; 1.2680x over previous
import functools
import math

import jax
import jax.numpy as jnp
from jax import lax
from jax.experimental import pallas as pl
from jax.experimental.pallas import tpu as pltpu
from jax.experimental.pallas import tpu_sc as plsc

D_MODEL = 64
SCALE = math.sqrt(D_MODEL)

NC = 2
NS = 16
NW = NC * NS
L = 16
VECS = D_MODEL // L

NBUF = 4
PIECES = ((0, 96), (96, 104))
MAXLEN = 104


def _make_lookup(R: int, C: int):
    assert R % NW == 0 and C == sum(p[1] for p in PIECES)
    rows_per_w = R // NW
    n_pieces = rows_per_w * 2
    assert n_pieces % NBUF == 0 and n_pieces // NBUF >= 3

    mesh = plsc.VectorSubcoreMesh(core_axis_name="c", subcore_axis_name="s")

    @functools.partial(
        pl.kernel,
        out_type=jax.ShapeDtypeStruct((R, C, D_MODEL), jnp.float32),
        mesh=mesh,
        compiler_params=pltpu.CompilerParams(use_tc_tiling_on_sc=False),
        scratch_types=[
            pltpu.VMEM((rows_per_w, C), jnp.int32),
            [pltpu.VMEM((MAXLEN, D_MODEL), jnp.float32)] * NBUF,
            [pltpu.VMEM((MAXLEN, D_MODEL), jnp.float32)] * NBUF,
            [pltpu.SemaphoreType.DMA] * NBUF,
            [pltpu.SemaphoreType.DMA] * NBUF,
        ],
    )
    def lookup(x_hbm, table_hbm, out_hbm, idx_v, gbufs, wbufs, gsems, wsems):
        wid = lax.axis_index("s") * NC + lax.axis_index("c")
        base_row = wid * rows_per_w
        pltpu.sync_copy(x_hbm.at[pl.ds(base_row, rows_per_w)], idx_v)

        def start_gather(b, p):
            off, ln = PIECES[b % 2]
            r = p // 2
            pltpu.async_copy(
                table_hbm.at[idx_v.at[r, pl.ds(off, ln)]],
                gbufs[b].at[pl.ds(0, ln)],
                gsems[b],
            )

        def gather_done(b):
            _, ln = PIECES[b % 2]
            pltpu.make_async_copy(
                table_hbm.at[pl.ds(0, ln)], gbufs[b].at[pl.ds(0, ln)], gsems[b]
            ).wait()

        def write_done(b):
            _, ln = PIECES[b % 2]
            pltpu.make_async_copy(
                wbufs[b].at[pl.ds(0, ln)],
                out_hbm.at[0, pl.ds(PIECES[b % 2][0], ln)],
                wsems[b],
            ).wait()

        def scale(b):
            _, ln = PIECES[b % 2]

            @plsc.parallel_loop(0, ln, step=1, unroll=4)
            def _row(r2):
                for v in range(VECS):
                    sl = pl.ds(v * L, L)
                    wbufs[b][r2, sl] = gbufs[b][r2, sl] * SCALE

        def start_write(b, p):
            off, ln = PIECES[b % 2]
            r = p // 2
            pltpu.async_copy(
                wbufs[b].at[pl.ds(0, ln)],
                out_hbm.at[base_row + r, pl.ds(off, ln)],
                wsems[b],
            )

        for b in range(NBUF):
            start_gather(b, b)

        @pl.loop(0, n_pieces - NBUF, step=NBUF)
        def _outer(p0):
            for b in range(NBUF):
                p = p0 + b
                gather_done(b)

                @pl.when(p0 > 0)
                def _():
                    write_done(b)

                scale(b)
                start_write(b, p)
                start_gather(b, p + NBUF)

        for b in range(NBUF):
            p = n_pieces - NBUF + b
            gather_done(b)
            write_done(b)
            scale(b)
            start_write(b, p)
        for b in range(NBUF):
            write_done(b)

    return lookup


@jax.jit
def kernel(x, table):
    return _make_lookup(x.shape[0], x.shape[1])(x, table)
